# baseline (speedup 1.0000x reference)
import functools

import jax
import jax.numpy as jnp
from jax import lax
from jax.experimental import pallas as pl
from jax.experimental.pallas import tpu as pltpu
from jax.experimental.pallas import tpu_sc as plsc

B = 16384
HID = 64
TXT_L = 50
FEAT_D = 128
CAT_V = 1000000

NC, NS = 2, 16
NW = NC * NS
BPW = B // NW
CHUNK = 128
NCK = BPW // CHUNK
LANES = 16

_MESH = plsc.VectorSubcoreMesh(core_axis_name="c", subcore_axis_name="s",
                               num_cores=NC, num_subcores=NS)


@functools.partial(
    pl.kernel,
    out_type=jax.ShapeDtypeStruct((B, HID), jnp.float32),
    mesh=_MESH,
    compiler_params=pltpu.CompilerParams(use_tc_tiling_on_sc=False),
    scratch_types=[
        pltpu.VMEM((TXT_L, NCK, CHUNK), jnp.int32),
        pltpu.VMEM((BPW, HID), jnp.float32),
        pltpu.SemaphoreType.DMA,
        pltpu.SemaphoreType.DMA,
    ],
)
def _sc_text(txt_hbm, ttxt_hbm, out_hbm, idx_t, acc, sem_idx, sem_txt):
  wid = lax.axis_index("s") * NC + lax.axis_index("c")
  base = wid * BPW

  pltpu.async_copy(txt_hbm.at[:, pl.ds(wid * NCK, NCK), :], idx_t,
                   sem_idx).wait()

  def fire(l, add):
    return [
        pltpu.async_copy(ttxt_hbm.at[idx_t.at[l, c]],
                         acc.at[pl.ds(c * CHUNK, CHUNK)], sem_txt, add=add)
        for c in range(NCK)
    ]

  for d in fire(0, False):
    d.wait()

  def txt_body(l, carry):
    for d in fire(l, True):
      d.wait()
    return carry
  lax.fori_loop(1, TXT_L, txt_body, 0)

  pltpu.sync_copy(acc, out_hbm.at[pl.ds(base, BPW)])


@functools.partial(
    pl.kernel,
    out_type=jax.ShapeDtypeStruct((B, 2 * HID), jnp.float32),
    mesh=_MESH,
    scratch_types=[
        pltpu.VMEM((NCK, CHUNK), jnp.int32),
        pltpu.VMEM((NCK, CHUNK), jnp.int32),
        pltpu.VMEM((BPW, 2 * HID), jnp.float32),
        pltpu.SemaphoreType.DMA,
        pltpu.SemaphoreType.DMA,
    ],
)
def _sc_cat(cat_hbm, tcat_hbm, out_hbm, idx_c, pid_c, rows, sem_idx,
            sem_cat):
  wid = lax.axis_index("s") * NC + lax.axis_index("c")
  base = wid * BPW

  pltpu.async_copy(cat_hbm.at[pl.ds(wid * NCK, NCK), :], idx_c,
                   sem_idx).wait()
  for g in range(NCK):
    for k in range(CHUNK // LANES):
      sl = pl.ds(k * LANES, LANES)
      v = idx_c[g, sl]
      pid_c[g, sl] = jnp.where(v >= CAT_V // 2, v - CAT_V // 2, v)

  descs = [
      pltpu.async_copy(tcat_hbm.at[pid_c.at[c]],
                       rows.at[pl.ds(c * CHUNK, CHUNK)], sem_cat)
      for c in range(NCK)
  ]
  for d in descs:
    d.wait()
  pltpu.sync_copy(rows, out_hbm.at[pl.ds(base, BPW)])


_BLK = 2048
_HALF = B // _BLK


def _tc_combine(text_sum, cat_pairs, cat_ids, len_col, user_feat, w_feat,
                b_feat):
  def body(text_ref, pair_ref, ids_ref, len_ref, x_ref, w_ref, b_ref, o_ref):
    i = pl.program_id(0)

    @pl.when(i < _HALF)
    def _item():
      odd = ids_ref[...] >= CAT_V // 2
      cat = jnp.where(odd, pair_ref[:, HID:], pair_ref[:, :HID])
      o_ref[...] = cat + text_ref[...] / len_ref[...]

    @pl.when(i >= _HALF)
    def _user():
      o_ref[...] = lax.dot_general(
          x_ref[...], w_ref[...], (((1,), (1,)), ((), ())),
          preferred_element_type=jnp.float32) + b_ref[...]

  return pl.pallas_call(
      body,
      grid=(2 * _HALF,),
      in_specs=[
          pl.BlockSpec((_BLK, HID), lambda i: (jnp.minimum(i, _HALF - 1), 0)),
          pl.BlockSpec((_BLK, 2 * HID),
                       lambda i: (jnp.minimum(i, _HALF - 1), 0)),
          pl.BlockSpec((_BLK, 1), lambda i: (jnp.minimum(i, _HALF - 1), 0)),
          pl.BlockSpec((_BLK, 1), lambda i: (jnp.minimum(i, _HALF - 1), 0)),
          pl.BlockSpec((_BLK, FEAT_D),
                       lambda i: (jnp.maximum(i - _HALF, 0), 0)),
          pl.BlockSpec((HID, FEAT_D), lambda i: (0, 0)),
          pl.BlockSpec((1, HID), lambda i: (0, 0)),
      ],
      out_specs=pl.BlockSpec((_BLK, HID), lambda i: (i, 0)),
      out_shape=jax.ShapeDtypeStruct((2 * B, HID), jnp.float32),
  )(text_sum, cat_pairs, cat_ids, len_col, user_feat, w_feat,
    b_feat.reshape(1, HID))


def kernel(item_cat, item_text, text_len, user_feat, table_cat, table_text,
           W_feat, b_feat):
  cat_i32 = item_cat.astype(jnp.int32)
  cat_idx = cat_i32.reshape(NW * NCK, CHUNK)
  text_t = item_text.astype(jnp.int32).T.reshape(TXT_L, NW * NCK, CHUNK)
  len_col = text_len.astype(jnp.float32).reshape(B, 1)
  tcat2 = jnp.concatenate(
      [table_cat[:CAT_V // 2], table_cat[CAT_V // 2:]], axis=1)
  text_sum = _sc_text(text_t, table_text)
  cat_pairs = _sc_cat(cat_idx, tcat2)
  return _tc_combine(text_sum, cat_pairs, cat_i32.reshape(B, 1), len_col,
                     user_feat, W_feat, b_feat)


# speedup vs baseline: 1.6938x; 1.6938x over previous
import functools

import jax
import jax.numpy as jnp
from jax import lax
from jax.experimental import layout
from jax.experimental import pallas as pl
from jax.experimental.pallas import tpu as pltpu
from jax.experimental.pallas import tpu_sc as plsc

B = 16384
HID = 64
TXT_L = 50
FEAT_D = 128
CAT_V = 1000000

NC, NS = 2, 16
NW = NC * NS
BPW = B // NW
CHUNK = 128
NCK = BPW // CHUNK
LANES = 16

_MESH = plsc.VectorSubcoreMesh(core_axis_name="c", subcore_axis_name="s",
                               num_cores=NC, num_subcores=NS)


@functools.partial(
    pl.kernel,
    out_type=jax.ShapeDtypeStruct((B, HID), jnp.float32),
    mesh=_MESH,
    compiler_params=pltpu.CompilerParams(use_tc_tiling_on_sc=False),
    scratch_types=[
        pltpu.VMEM((TXT_L, NCK, CHUNK), jnp.int32),
        pltpu.VMEM((BPW, HID), jnp.float32),
        pltpu.SemaphoreType.DMA,
        pltpu.SemaphoreType.DMA,
    ],
)
def _sc_text(txt_hbm, ttxt_hbm, out_hbm, idx_t, acc, sem_idx, sem_txt):
  wid = lax.axis_index("s") * NC + lax.axis_index("c")
  base = wid * BPW

  pltpu.async_copy(txt_hbm.at[:, pl.ds(wid * NCK, NCK), :], idx_t,
                   sem_idx).wait()

  def fire(l, add):
    return [
        pltpu.async_copy(ttxt_hbm.at[idx_t.at[l, c]],
                         acc.at[pl.ds(c * CHUNK, CHUNK)], sem_txt, add=add)
        for c in range(NCK)
    ]

  for d in fire(0, False):
    d.wait()

  def txt_body(l, carry):
    for d in fire(l, True):
      d.wait()
    return carry
  lax.fori_loop(1, TXT_L, txt_body, 0)

  pltpu.sync_copy(acc, out_hbm.at[pl.ds(base, BPW)])


@functools.partial(
    pl.kernel,
    out_type=jax.ShapeDtypeStruct((B, HID), jnp.float32),
    mesh=_MESH,
    compiler_params=pltpu.CompilerParams(use_tc_tiling_on_sc=False),
    scratch_types=[
        pltpu.VMEM((NCK, CHUNK), jnp.int32),
        pltpu.VMEM((BPW, HID), jnp.float32),
        pltpu.SemaphoreType.DMA,
        pltpu.SemaphoreType.DMA,
    ],
)
def _sc_cat(cat_hbm, tcat_hbm, out_hbm, idx_c, rows, sem_idx, sem_cat):
  wid = lax.axis_index("s") * NC + lax.axis_index("c")
  base = wid * BPW

  pltpu.async_copy(cat_hbm.at[pl.ds(wid * NCK, NCK), :], idx_c,
                   sem_idx).wait()
  descs = [
      pltpu.async_copy(tcat_hbm.at[idx_c.at[c]],
                       rows.at[pl.ds(c * CHUNK, CHUNK)], sem_cat)
      for c in range(NCK)
  ]
  for d in descs:
    d.wait()
  pltpu.sync_copy(rows, out_hbm.at[pl.ds(base, BPW)])


_BLK = 2048
_HALF = B // _BLK


def _tc_combine(text_sum, cat_rows, len_col, user_feat, w_feat, b_feat):
  def body(text_ref, cat_ref, len_ref, x_ref, w_ref, b_ref, o_ref):
    i = pl.program_id(0)

    @pl.when(i < _HALF)
    def _item():
      o_ref[...] = cat_ref[...] + text_ref[...] / len_ref[...]

    @pl.when(i >= _HALF)
    def _user():
      o_ref[...] = lax.dot_general(
          x_ref[...], w_ref[...], (((1,), (1,)), ((), ())),
          preferred_element_type=jnp.float32) + b_ref[...]

  return pl.pallas_call(
      body,
      grid=(2 * _HALF,),
      in_specs=[
          pl.BlockSpec((_BLK, HID), lambda i: (jnp.minimum(i, _HALF - 1), 0)),
          pl.BlockSpec((_BLK, HID), lambda i: (jnp.minimum(i, _HALF - 1), 0)),
          pl.BlockSpec((_BLK, 1), lambda i: (jnp.minimum(i, _HALF - 1), 0)),
          pl.BlockSpec((_BLK, FEAT_D),
                       lambda i: (jnp.maximum(i - _HALF, 0), 0)),
          pl.BlockSpec((HID, FEAT_D), lambda i: (0, 0)),
          pl.BlockSpec((1, HID), lambda i: (0, 0)),
      ],
      out_specs=pl.BlockSpec((_BLK, HID), lambda i: (i, 0)),
      out_shape=jax.ShapeDtypeStruct((2 * B, HID), jnp.float32),
  )(text_sum, cat_rows, len_col, user_feat, w_feat, b_feat.reshape(1, HID))


def kernel(item_cat, item_text, text_len, user_feat, table_cat, table_text,
           W_feat, b_feat):
  cat_i32 = item_cat.astype(jnp.int32)
  cat_idx = cat_i32.reshape(NW * NCK, CHUNK)
  text_t = item_text.astype(jnp.int32).T.reshape(TXT_L, NW * NCK, CHUNK)
  len_col = text_len.astype(jnp.float32).reshape(B, 1)
  tcat_lin = layout.with_layout_constraint(
      table_cat, layout.Layout((0, 1), tiling=((8,),)))
  text_sum = _sc_text(text_t, table_text)
  cat_rows = _sc_cat(cat_idx, tcat_lin)
  return _tc_combine(text_sum, cat_rows, len_col, user_feat, W_feat, b_feat)
